# Initial kernel scaffold; baseline (speedup 1.0000x reference)
import jax
import jax.numpy as jnp
from jax.experimental import pallas as pl


def kernel(x, edge_index, W_self, W_neigh, bias):
    raise NotImplementedError("write your pallas kernel here")


# speedup vs baseline: 9.9550x; 9.9550x over previous
import functools

import jax
import jax.numpy as jnp
from jax import lax
from jax.experimental import pallas as pl
from jax.experimental.pallas import tpu as pltpu
from jax.experimental.pallas import tpu_sc as plsc

NC = 2
NS = 16
NW = NC * NS
C = 128


def _sc_agg_kernel(n, d, k, n_acc):
    zr = n_acc // NS

    out_type = jax.ShapeDtypeStruct((NC, n_acc, d), jnp.float32)
    scratch = [
        pltpu.VMEM_SHARED((n_acc, d), jnp.float32),
        pltpu.VMEM((2, C), jnp.int32),
        pltpu.VMEM((2, C), jnp.int32),
        pltpu.VMEM((C, d), jnp.float32),
        pltpu.VMEM((C, d), jnp.float32),
        pltpu.SemaphoreType.DMA,
        pltpu.SemaphoreType.DMA,
        pltpu.SemaphoreType.DMA,
        pltpu.SemaphoreType.DMA,
    ]

    mesh = plsc.VectorSubcoreMesh(core_axis_name="c", subcore_axis_name="s",
                                  num_cores=NC, num_subcores=NS)

    def body(h_hbm, sd_hbm, zrows_hbm, agg_out, acc, i0, i1, rows0, rows1,
             sem0, sem1, semi0, semi1):
        c = lax.axis_index("c")
        s = lax.axis_index("s")
        wid = c * NS + s

        pltpu.sync_copy(zrows_hbm, acc.at[pl.ds(s * zr, zr)])
        plsc.subcore_barrier()

        def idx_start(j, buf, sem):
            pltpu.async_copy(sd_hbm.at[wid, j], buf, sem)

        def idx_wait(buf, sem):
            pltpu.make_async_copy(sd_hbm.at[wid, 0], buf, sem).wait()

        def gather_start(ibuf, buf, sem):
            pltpu.async_copy(h_hbm.at[ibuf.at[0]], buf, sem)

        def gather_wait(buf, sem):
            pltpu.make_async_copy(h_hbm.at[i0.at[0]], buf, sem).wait()

        def scatter(ibuf, buf):
            pltpu.sync_copy(buf, acc.at[ibuf.at[1]], add=True)

        pltpu.sync_copy(sd_hbm.at[wid, 0], i0)
        gather_start(i0, rows0, sem0)
        idx_start(1, i1, semi1)

        def step(jj, carry):
            j0 = jj * 2
            more = jj + 1 < k // 2
            gather_wait(rows0, sem0)
            idx_wait(i1, semi1)
            gather_start(i1, rows1, sem1)
            scatter(i0, rows0)

            @pl.when(more)
            def _():
                idx_start(j0 + 2, i0, semi0)

            gather_wait(rows1, sem1)

            @pl.when(more)
            def _():
                idx_wait(i0, semi0)
                gather_start(i0, rows0, sem0)

            scatter(i1, rows1)

            @pl.when(more)
            def _():
                idx_start(j0 + 3, i1, semi1)

            return carry

        lax.fori_loop(0, k // 2, step, 0)
        plsc.subcore_barrier()

        pltpu.sync_copy(acc.at[pl.ds(s * zr, zr)], agg_out.at[c, pl.ds(s * zr, zr)])

    return pl.kernel(body, out_type=out_type, mesh=mesh,
                     scratch_types=scratch)


def _sc_deg_kernel(k, n_acc):
    rows = n_acc // 128
    out_type = jax.ShapeDtypeStruct((NC, rows, 128), jnp.float32)
    scratch = [
        pltpu.VMEM_SHARED((rows, 128), jnp.float32),
        pltpu.VMEM((rows, 128), jnp.float32),
        pltpu.VMEM((2, C), jnp.int32),
        pltpu.VMEM((2, C), jnp.int32),
        pltpu.VMEM((rows,), jnp.int32),
        pltpu.SemaphoreType.DMA,
        pltpu.SemaphoreType.DMA,
    ]
    mesh = plsc.VectorSubcoreMesh(core_axis_name="c", subcore_axis_name="s",
                                  num_cores=NC, num_subcores=NS)

    def body(sd_hbm, deg_out, dacc, hist, i0, i1, idn, semi0, semi1):
        c = lax.axis_index("c")
        s = lax.axis_index("s")
        wid = c * NS + s

        z16 = jnp.zeros((16,), jnp.float32)

        def zrow(r, carry):
            for cc in range(8):
                hist[r, pl.ds(cc * 16, 16)] = z16
            return carry

        lax.fori_loop(0, rows, zrow, 0)

        def irow(r, carry):
            idn[pl.ds(r * 16, 16)] = lax.iota(jnp.int32, 16) + r * 16
            return carry

        lax.fori_loop(0, rows // 16, irow, 0)

        @pl.when(s == 0)
        def _():
            pltpu.sync_copy(hist, dacc)
        plsc.subcore_barrier()

        def idx_start(j, buf, sem):
            pltpu.async_copy(sd_hbm.at[wid, j], buf, sem)

        def idx_wait(buf, sem):
            pltpu.make_async_copy(sd_hbm.at[wid, 0], buf, sem).wait()

        ones = jnp.ones((16,), jnp.float32)

        def chunk(ibuf):
            for r in range(C // 16):
                v = ibuf[1, pl.ds(r * 16, 16)]
                vhi = lax.shift_right_logical(v, 7)
                vlo = lax.bitwise_and(v, 127)
                plsc.addupdate_scatter(hist, [vhi, vlo], ones)

        idx_start(0, i0, semi0)

        def step(jj, carry):
            j0 = jj * 2
            more = jj + 1 < k // 2
            idx_wait(i0, semi0)
            idx_start(j0 + 1, i1, semi1)
            chunk(i0)
            idx_wait(i1, semi1)

            @pl.when(more)
            def _():
                idx_start(j0 + 2, i0, semi0)

            chunk(i1)
            return carry

        lax.fori_loop(0, k // 2, step, 0)

        pltpu.sync_copy(hist, dacc.at[idn], add=True)
        plsc.subcore_barrier()

        @pl.when(s < rows // 8)
        def _():
            pltpu.sync_copy(dacc.at[pl.ds(s * 8, 8)],
                            deg_out.at[c, pl.ds(s * 8, 8)])

    return pl.kernel(body, out_type=out_type, mesh=mesh,
                     scratch_types=scratch,
                     compiler_params=pltpu.CompilerParams(
                         needs_layout_passes=False))


def _dense_body(h_ref, agg_ref, deg_ref, ws_ref, wn_ref, b_ref, out_ref):
    deg = jnp.maximum(deg_ref[...], 1.0)
    agg = (agg_ref[0] + agg_ref[1]) / deg
    hn = jnp.dot(agg, wn_ref[...], preferred_element_type=jnp.float32)
    hs = jnp.dot(h_ref[...], ws_ref[...], preferred_element_type=jnp.float32)
    h = jnp.maximum(hs + hn + b_ref[...], 0.0)
    nrm = jnp.maximum(jnp.sqrt(jnp.sum(h * h, axis=1, keepdims=True)), 1e-12)
    out_ref[...] = h / nrm


def _dense_layer(h, agg, deg, w_self, w_neigh, bias_row, br=400):
    n, d = h.shape
    grid = (n // br,)
    return pl.pallas_call(
        _dense_body,
        grid=grid,
        in_specs=[
            pl.BlockSpec((br, d), lambda i: (i, 0)),
            pl.BlockSpec((NC, br, d), lambda i: (0, i, 0)),
            pl.BlockSpec((br, 1), lambda i: (i, 0)),
            pl.BlockSpec((d, d), lambda i: (0, 0)),
            pl.BlockSpec((d, d), lambda i: (0, 0)),
            pl.BlockSpec((1, d), lambda i: (0, 0)),
        ],
        out_specs=pl.BlockSpec((br, d), lambda i: (i, 0)),
        out_shape=jax.ShapeDtypeStruct((n, d), jnp.float32),
    )(h, agg, deg, w_self, w_neigh, bias_row)


def kernel(x, edge_index, W_self, W_neigh, bias):
    n, d = x.shape
    e = edge_index.shape[1]
    L = W_self.shape[0]

    k = -(-e // (NW * C))
    k += k % 2
    e_pad = k * NW * C
    n_acc = -(-(n + 128) // 128) * 128
    npad_rows = n_acc - n

    src = edge_index[0]
    dst = edge_index[1]
    pad = e_pad - e
    if pad:
        ar = jnp.arange(pad, dtype=jnp.int32)
        src = jnp.concatenate([src, ar % n])
        dst = jnp.concatenate([dst, n + (ar % npad_rows)])
    sd = jnp.stack([src.reshape(NW, k, C), dst.reshape(NW, k, C)], axis=2)

    zrows = jnp.zeros((n_acc // NS, d), jnp.float32)

    agg_fn = _sc_agg_kernel(n, d, k, n_acc)
    deg_fn = _sc_deg_kernel(k, n_acc)

    degp = deg_fn(sd)
    deg = (degp[0] + degp[1]).reshape(n_acc)[:n, None]
    h = x
    for l in range(L):
        agg = agg_fn(h, sd, zrows)
        h = _dense_layer(h, agg, deg, W_self[l], W_neigh[l], bias[l][None, :])
    return h
